# Initial kernel scaffold; baseline (speedup 1.0000x reference)
import jax
import jax.numpy as jnp
from jax.experimental import pallas as pl


def kernel(x, Wg, W1, W2):
    raise NotImplementedError("write your pallas kernel here")


# speedup vs baseline: 1.2135x; 1.2135x over previous
import functools

import jax
import jax.numpy as jnp
from jax.experimental import pallas as pl
from jax.experimental.pallas import tpu as pltpu

B, S, D = 1, 2048, 1024
E, K, H = 8, 2, 1024
N = B * S
TT = 256
NT = N // TT
NEG = -1e30


def _router_body(wg_ref, x_ref, logits_ref, comb_ref):
    xt = x_ref[...]
    lt = jax.lax.dot_general(
        wg_ref[...], xt, (((1,), (1,)), ((), ())),
        preferred_element_type=jnp.float32)
    idx = jax.lax.broadcasted_iota(jnp.int32, (E, TT), 0)
    m1 = jnp.max(lt, axis=0, keepdims=True)
    a1 = jnp.min(jnp.where(lt == m1, idx, E), axis=0, keepdims=True)
    sel1 = idx == a1
    lt2 = jnp.where(sel1, NEG, lt)
    m2 = jnp.max(lt2, axis=0, keepdims=True)
    a2 = jnp.min(jnp.where(lt2 == m2, idx, E), axis=0, keepdims=True)
    sel2 = idx == a2
    e2 = jnp.exp(m2 - m1)
    denom = 1.0 + e2
    w1 = 1.0 / denom
    w2 = e2 / denom
    comb = jnp.where(sel1, w1, 0.0) + jnp.where(sel2, w2, 0.0)
    logits_ref[...] = lt
    comb_ref[...] = comb


def _router(xf, Wg):
    return pl.pallas_call(
        _router_body,
        grid=(NT,),
        in_specs=[
            pl.BlockSpec((E, D), lambda t: (0, 0)),
            pl.BlockSpec((TT, D), lambda t: (t, 0)),
        ],
        out_specs=[
            pl.BlockSpec((E, TT), lambda t: (0, t)),
            pl.BlockSpec((E, TT), lambda t: (0, t)),
        ],
        out_shape=[
            jax.ShapeDtypeStruct((E, N), jnp.float32),
            jax.ShapeDtypeStruct((E, N), jnp.float32),
        ],
    )(Wg, xf)


def _expert_body(x_ref, w1_ref, w2_ref, comb_ref, out_ref):
    e = pl.program_id(1)
    xt = x_ref[...]
    g = jax.lax.dot_general(
        xt, w1_ref[0, 0], (((1,), (1,)), ((), ())),
        preferred_element_type=jnp.float32)
    l = jax.lax.dot_general(
        xt, w1_ref[0, 1], (((1,), (1,)), ((), ())),
        preferred_element_type=jnp.float32)
    act = g * jax.lax.logistic(g) * l
    oe = jax.lax.dot_general(
        act, w2_ref[0], (((1,), (1,)), ((), ())),
        preferred_element_type=jnp.float32)
    eidx = jax.lax.broadcasted_iota(jnp.int32, (TT, E), 1)
    ccol = jnp.sum(jnp.where(eidx == e, comb_ref[...], 0.0),
                   axis=1, keepdims=True)
    contrib = ccol * oe

    @pl.when(e == 0)
    def _():
        out_ref[...] = contrib

    @pl.when(e > 0)
    def _():
        out_ref[...] += contrib


def _experts(xf, W1r, W2, comb):
    return pl.pallas_call(
        _expert_body,
        grid=(NT, E),
        in_specs=[
            pl.BlockSpec((TT, D), lambda t, e: (t, 0)),
            pl.BlockSpec((1, 2, H, D), lambda t, e: (e, 0, 0, 0)),
            pl.BlockSpec((1, D, H), lambda t, e: (e, 0, 0)),
            pl.BlockSpec((TT, E), lambda t, e: (t, 0)),
        ],
        out_specs=pl.BlockSpec((TT, D), lambda t, e: (t, 0)),
        out_shape=jax.ShapeDtypeStruct((N, D), jnp.float32),
    )(xf, W1r, W2, comb)


@jax.jit
def kernel(x, Wg, W1, W2):
    xf = x.reshape(N, D)
    logitsT, combT = _router(xf, Wg)
    comb = combT.T
    W1r = W1.reshape(E, 2, H, D)
    out = _experts(xf, W1r, W2, comb)
    return out.reshape(B, S, D), logitsT.T.reshape(B, S, E)
